# Initial kernel scaffold; baseline (speedup 1.0000x reference)
import jax
import jax.numpy as jnp
from jax.experimental import pallas as pl


def kernel(roi_bboxes, gt_boxes, gt_labels):
    raise NotImplementedError("write your pallas kernel here")


# speedup vs baseline: 1.0735x; 1.0735x over previous
import functools

import jax
import jax.numpy as jnp
from jax.experimental import pallas as pl
from jax.experimental.pallas import tpu as pltpu

_L = 21
_TOT_POS = 128
_TOT_NEG = 128
_VARS = (0.1, 0.1, 0.2, 0.2)


def _select(mask, ranks, total, n_pad):
    def body(_, c):
        lo, hi = c
        mid = (lo + hi) // 2
        cnt = jnp.sum(jnp.where(mask & (ranks <= mid), 1, 0))
        ok = (cnt >= total) | (mid == n_pad - 1)
        return jnp.where(ok, lo, mid + 1), jnp.where(ok, mid, hi)

    _, t = jax.lax.fori_loop(
        0, 14, body, (jnp.int32(0), jnp.int32(n_pad - 1)))
    return mask & (ranks <= t)


def _body(m_real, m_pad, n_pad, roi_ref, gt_ref, gtl_ref, rp_ref, rn_ref,
          od_ref, ol_ref):
    f32 = jnp.float32
    roi = roi_ref[0]
    gt_t = gt_ref[0]
    gtl_t = gtl_ref[0]
    rp = rp_ref[0]
    rn = rn_ref[0]

    b_y1 = roi[:, 0:1]
    b_x1 = roi[:, 1:2]
    b_y2 = roi[:, 2:3]
    b_x2 = roi[:, 3:4]
    g_y1 = gt_t[0:1, :]
    g_x1 = gt_t[1:2, :]
    g_y2 = gt_t[2:3, :]
    g_x2 = gt_t[3:4, :]

    area_b = (b_y2 - b_y1) * (b_x2 - b_x1)
    area_g = (g_y2 - g_y1) * (g_x2 - g_x1)
    x_top = jnp.maximum(b_x1, g_x1)
    y_top = jnp.maximum(b_y1, g_y1)
    x_bot = jnp.minimum(b_x2, g_x2)
    y_bot = jnp.minimum(b_y2, g_y2)
    inter = jnp.maximum(x_bot - x_top, 0.0) * jnp.maximum(y_bot - y_top, 0.0)
    union = area_b + area_g - inter
    iou = inter / union
    col = jax.lax.broadcasted_iota(jnp.int32, (n_pad, m_pad), 1)
    iou = jnp.where(col < m_real, iou, -1.0)

    merged = jnp.max(iou, axis=1, keepdims=True)
    idx = jnp.min(jnp.where(iou == merged, col, m_pad), axis=1,
                  keepdims=True)
    eqc = col == idx

    def pick(row):
        return jnp.sum(jnp.where(eqc, row, 0.0), axis=1, keepdims=True)

    glab = pick(gtl_t)

    pos_pre = merged > 0.5
    neg_pre = (merged < 0.5) & (merged > 0.1)
    sel_pos = _select(pos_pre, rp, _TOT_POS, n_pad)
    sel_neg = _select(neg_pre, rn, _TOT_NEG, n_pad)

    lab = (jnp.where(sel_pos, glab, -1.0)
           + jnp.where(sel_neg, 1.0, 0.0))

    bw0 = b_x2 - b_x1
    bh0 = b_y2 - b_y1
    bcx = b_x1 + 0.5 * bw0
    bcy = b_y1 + 0.5 * bh0
    gy1 = jnp.where(sel_pos, pick(g_y1), 0.0)
    gx1 = jnp.where(sel_pos, pick(g_x1), 0.0)
    gy2 = jnp.where(sel_pos, pick(g_y2), 0.0)
    gx2 = jnp.where(sel_pos, pick(g_x2), 0.0)
    gw = gx2 - gx1
    gh = gy2 - gy1
    gcx = gx1 + 0.5 * gw
    gcy = gy1 + 0.5 * gh
    bw = jnp.where(bw0 == 0, 1e-3, bw0)
    bh = jnp.where(bh0 == 0, 1e-3, bh0)
    gws = jnp.where(gw == 0, 1.0, gw)
    ghs = jnp.where(gh == 0, 1.0, gh)
    dy = jnp.where(gh == 0, 0.0, (gcy - bcy) / bh) / _VARS[0]
    dx = jnp.where(gw == 0, 0.0, (gcx - bcx) / bw) / _VARS[1]
    dh = jnp.where(gh == 0, 0.0, jnp.log(ghs / bh)) / _VARS[2]
    dw = jnp.where(gw == 0, 0.0, jnp.log(gws / bw)) / _VARS[3]

    c_i = jax.lax.broadcasted_iota(jnp.int32, (n_pad, 4 * _L), 1)
    l_i = (c_i // 4).astype(f32)
    k_i = c_i % 4
    a = (l_i == lab).astype(f32)
    bmat = (jnp.where(k_i == 0, dy, 0.0) + jnp.where(k_i == 1, dx, 0.0)
            + jnp.where(k_i == 2, dh, 0.0) + jnp.where(k_i == 3, dw, 0.0))
    od_ref[0] = a * bmat

    l21 = jax.lax.broadcasted_iota(jnp.int32, (n_pad, _L), 1).astype(f32)
    ol_ref[0] = (l21 == lab).astype(f32)


def _ranks(key, b, n, n_pad, total):
    r = jax.random.randint(key, (b, n), 1, total * 10)
    order = jnp.argsort(-r, axis=-1)
    rank = jnp.argsort(order, axis=-1).astype(jnp.int32)
    return jnp.pad(rank, ((0, 0), (0, n_pad - n)),
                   constant_values=n_pad - 1)[..., None]


def kernel(roi_bboxes, gt_boxes, gt_labels):
    b, n, _ = roi_bboxes.shape
    m = gt_boxes.shape[1]
    n_pad = ((n + 127) // 128) * 128
    m_pad = 128

    roi_p = jnp.pad(roi_bboxes, ((0, 0), (0, n_pad - n), (0, 0)))
    gt_t = jnp.pad(jnp.swapaxes(gt_boxes, 1, 2),
                   ((0, 0), (0, 0), (0, m_pad - m)))
    gtl_t = jnp.pad(gt_labels.astype(jnp.float32)[:, None, :],
                    ((0, 0), (0, 0), (0, m_pad - m)))
    rp = _ranks(jax.random.key(11), b, n, n_pad, _TOT_POS)
    rn = _ranks(jax.random.key(13), b, n, n_pad, _TOT_NEG)

    out_shape = (
        jax.ShapeDtypeStruct((b, n_pad, 4 * _L), jnp.float32),
        jax.ShapeDtypeStruct((b, n_pad, _L), jnp.float32),
    )
    deltas, labels = pl.pallas_call(
        functools.partial(_body, m, m_pad, n_pad),
        grid=(b,),
        in_specs=[
            pl.BlockSpec((1, n_pad, 4), lambda i: (i, 0, 0)),
            pl.BlockSpec((1, 4, m_pad), lambda i: (i, 0, 0)),
            pl.BlockSpec((1, 1, m_pad), lambda i: (i, 0, 0)),
            pl.BlockSpec((1, n_pad, 1), lambda i: (i, 0, 0)),
            pl.BlockSpec((1, n_pad, 1), lambda i: (i, 0, 0)),
        ],
        out_specs=(
            pl.BlockSpec((1, n_pad, 4 * _L), lambda i: (i, 0, 0)),
            pl.BlockSpec((1, n_pad, _L), lambda i: (i, 0, 0)),
        ),
        out_shape=out_shape,
        compiler_params=pltpu.CompilerParams(
            dimension_semantics=("arbitrary",),
            vmem_limit_bytes=100 * 1024 * 1024),
    )(roi_p, gt_t, gtl_t, rp, rn)

    return (deltas[:, :n, :].reshape(b, n, _L, 4), labels[:, :n, :])


# speedup vs baseline: 1.9093x; 1.7786x over previous
import functools

import jax
import jax.numpy as jnp
import numpy as np
from jax.experimental import pallas as pl
from jax.experimental.pallas import tpu as pltpu

_L = 21
_TOT_POS = 128
_TOT_NEG = 128
_VARS = (0.1, 0.1, 0.2, 0.2)


def _select(mask, ranks, total, n_pad):
    def body(_, c):
        lo, hi = c
        mid = (lo + hi) // 2
        cnt = jnp.sum(jnp.where(mask & (ranks <= mid), 1, 0))
        ok = (cnt >= total) | (mid == n_pad - 1)
        return jnp.where(ok, lo, mid + 1), jnp.where(ok, mid, hi)

    _, t = jax.lax.fori_loop(
        0, 14, body, (jnp.int32(0), jnp.int32(n_pad - 1)))
    return mask & (ranks <= t)


def _body(m_real, m_pad, n_pad, roi_ref, gt_ref, gtl_ref, rp_ref, rn_ref,
          od_ref, ol_ref):
    f32 = jnp.float32
    roi = roi_ref[0]
    gt_t = gt_ref[0]
    gtl_t = gtl_ref[0]
    rp = rp_ref[0]
    rn = rn_ref[0]

    b_y1 = roi[:, 0:1]
    b_x1 = roi[:, 1:2]
    b_y2 = roi[:, 2:3]
    b_x2 = roi[:, 3:4]
    g_y1 = gt_t[0:1, :]
    g_x1 = gt_t[1:2, :]
    g_y2 = gt_t[2:3, :]
    g_x2 = gt_t[3:4, :]

    area_b = (b_y2 - b_y1) * (b_x2 - b_x1)
    area_g = (g_y2 - g_y1) * (g_x2 - g_x1)
    x_top = jnp.maximum(b_x1, g_x1)
    y_top = jnp.maximum(b_y1, g_y1)
    x_bot = jnp.minimum(b_x2, g_x2)
    y_bot = jnp.minimum(b_y2, g_y2)
    inter = jnp.maximum(x_bot - x_top, 0.0) * jnp.maximum(y_bot - y_top, 0.0)
    union = area_b + area_g - inter
    iou = inter / union
    col = jax.lax.broadcasted_iota(jnp.int32, (n_pad, m_pad), 1)
    iou = jnp.where(col < m_real, iou, -1.0)

    merged = jnp.max(iou, axis=1, keepdims=True)
    idx = jnp.min(jnp.where(iou == merged, col, m_pad), axis=1,
                  keepdims=True)
    eqc = col == idx

    def pick(row):
        return jnp.sum(jnp.where(eqc, row, 0.0), axis=1, keepdims=True)

    glab = pick(gtl_t)

    pos_pre = merged > 0.5
    neg_pre = (merged < 0.5) & (merged > 0.1)
    sel_pos = _select(pos_pre, rp, _TOT_POS, n_pad)
    sel_neg = _select(neg_pre, rn, _TOT_NEG, n_pad)

    lab = (jnp.where(sel_pos, glab, -1.0)
           + jnp.where(sel_neg, 1.0, 0.0))

    bw0 = b_x2 - b_x1
    bh0 = b_y2 - b_y1
    bcx = b_x1 + 0.5 * bw0
    bcy = b_y1 + 0.5 * bh0
    gy1 = jnp.where(sel_pos, pick(g_y1), 0.0)
    gx1 = jnp.where(sel_pos, pick(g_x1), 0.0)
    gy2 = jnp.where(sel_pos, pick(g_y2), 0.0)
    gx2 = jnp.where(sel_pos, pick(g_x2), 0.0)
    gw = gx2 - gx1
    gh = gy2 - gy1
    gcx = gx1 + 0.5 * gw
    gcy = gy1 + 0.5 * gh
    bw = jnp.where(bw0 == 0, 1e-3, bw0)
    bh = jnp.where(bh0 == 0, 1e-3, bh0)
    gws = jnp.where(gw == 0, 1.0, gw)
    ghs = jnp.where(gh == 0, 1.0, gh)
    dy = jnp.where(gh == 0, 0.0, (gcy - bcy) / bh) / _VARS[0]
    dx = jnp.where(gw == 0, 0.0, (gcx - bcx) / bw) / _VARS[1]
    dh = jnp.where(gh == 0, 0.0, jnp.log(ghs / bh)) / _VARS[2]
    dw = jnp.where(gw == 0, 0.0, jnp.log(gws / bw)) / _VARS[3]

    c_i = jax.lax.broadcasted_iota(jnp.int32, (n_pad, 4 * _L), 1)
    l_i = (c_i // 4).astype(f32)
    k_i = c_i % 4
    a = (l_i == lab).astype(f32)
    bmat = (jnp.where(k_i == 0, dy, 0.0) + jnp.where(k_i == 1, dx, 0.0)
            + jnp.where(k_i == 2, dh, 0.0) + jnp.where(k_i == 3, dw, 0.0))
    od_ref[0] = a * bmat

    l21 = jax.lax.broadcasted_iota(jnp.int32, (n_pad, _L), 1).astype(f32)
    ol_ref[0] = (l21 == lab).astype(f32)


_RANK_CACHE = {}


def _ranks(b, n):
    if (b, n) not in _RANK_CACHE:
        def one(seed, total):
            with jax.ensure_compile_time_eval():
                r = jax.random.randint(jax.random.key(seed), (b, n), 1,
                                       total * 10)
                order = jnp.argsort(-r, axis=-1)
                rank = jnp.argsort(order, axis=-1).astype(jnp.int32)
            return np.asarray(jax.device_get(rank))[..., None]
        _RANK_CACHE[(b, n)] = (one(11, _TOT_POS), one(13, _TOT_NEG))
    return _RANK_CACHE[(b, n)]


def kernel(roi_bboxes, gt_boxes, gt_labels):
    b, n, _ = roi_bboxes.shape
    m = gt_boxes.shape[1]
    m_pad = 128

    gt_t = jnp.pad(jnp.swapaxes(gt_boxes, 1, 2),
                   ((0, 0), (0, 0), (0, m_pad - m)))
    gtl_t = jnp.pad(gt_labels.astype(jnp.float32)[:, None, :],
                    ((0, 0), (0, 0), (0, m_pad - m)))
    rp_np, rn_np = _ranks(b, n)
    rp = jnp.asarray(rp_np)
    rn = jnp.asarray(rn_np)

    out_shape = (
        jax.ShapeDtypeStruct((b, n, 4 * _L), jnp.float32),
        jax.ShapeDtypeStruct((b, n, _L), jnp.float32),
    )
    deltas, labels = pl.pallas_call(
        functools.partial(_body, m, m_pad, n),
        grid=(b,),
        in_specs=[
            pl.BlockSpec((1, n, 4), lambda i: (i, 0, 0)),
            pl.BlockSpec((1, 4, m_pad), lambda i: (i, 0, 0)),
            pl.BlockSpec((1, 1, m_pad), lambda i: (i, 0, 0)),
            pl.BlockSpec((1, n, 1), lambda i: (i, 0, 0)),
            pl.BlockSpec((1, n, 1), lambda i: (i, 0, 0)),
        ],
        out_specs=(
            pl.BlockSpec((1, n, 4 * _L), lambda i: (i, 0, 0)),
            pl.BlockSpec((1, n, _L), lambda i: (i, 0, 0)),
        ),
        out_shape=out_shape,
        compiler_params=pltpu.CompilerParams(
            dimension_semantics=("arbitrary",),
            vmem_limit_bytes=100 * 1024 * 1024),
    )(roi_bboxes, gt_t, gtl_t, rp, rn)

    return (deltas.reshape(b, n, _L, 4), labels)
